# Initial kernel scaffold; baseline (speedup 1.0000x reference)
import functools

import jax
import jax.numpy as jnp
from jax import lax
from jax.experimental import pallas as pl
from jax.experimental.pallas import tpu as pltpu
from jax.experimental.pallas import tpu_sc as plsc

NC = 2
NS = 16
L = 16
NW = NC * NS

N = 4096
C = 4096
RPW = N // NW
RB = 8
NBATCH = RPW // RB


def _worker_id():
    return lax.axis_index("s") * NC + lax.axis_index("c")


_mesh = plsc.VectorSubcoreMesh(core_axis_name="c", subcore_axis_name="s")


@functools.partial(
    pl.kernel,
    out_type=jax.ShapeDtypeStruct((N,), jnp.float32),
    mesh=_mesh,
    scratch_types=[
        pltpu.VMEM((RB, C), jnp.int32),
        pltpu.VMEM((256,), jnp.int32),
        pltpu.VMEM((RPW,), jnp.float32),
    ],
)
def _den_kernel(x_hbm, dtab_hbm, den_hbm, xbuf, dtab, denout):
    wid = _worker_id()
    base_row = wid * RPW
    pltpu.sync_copy(dtab_hbm, dtab)

    def batch_body(b, _):
        row0 = base_row + b * RB
        pltpu.sync_copy(x_hbm.at[pl.ds(row0, RB)], xbuf)

        def row_body(rr, _):
            def grp_body(g, acc):
                xv = xbuf[rr, pl.ds(g * L, L)]
                return acc + plsc.load_gather(dtab, [xv])

            acc = lax.fori_loop(0, C // L, grp_body, jnp.zeros((L,), jnp.int32))
            denout[b * RB + rr] = jnp.sum(acc).astype(jnp.float32)
            return 0

        lax.fori_loop(0, RB, row_body, 0)
        return 0

    lax.fori_loop(0, NBATCH, batch_body, 0)
    pltpu.sync_copy(denout, den_hbm.at[pl.ds(base_row, RPW)])


@functools.partial(
    pl.kernel,
    out_type=jax.ShapeDtypeStruct((N, C // 4), jnp.int32),
    mesh=_mesh,
    scratch_types=[
        pltpu.VMEM((RB, C), jnp.int32),
        pltpu.VMEM((256,), jnp.int32),
        pltpu.VMEM((N,), jnp.float32),
        pltpu.VMEM((RB, C // 4), jnp.int32),
    ],
)
def _main_kernel(x_hbm, ntab_hbm, den_hbm, out_hbm, xbuf, ntab, den, outbuf):
    wid = _worker_id()
    base_row = wid * RPW
    pltpu.sync_copy(ntab_hbm, ntab)
    pltpu.sync_copy(den_hbm, den)
    iota = lax.iota(jnp.int32, L)

    def batch_body(b, _):
        row0 = base_row + b * RB
        pltpu.sync_copy(x_hbm.at[pl.ds(row0, RB)], xbuf)

        def row_body(rr, _):
            rsplat = jnp.full((L,), rr, jnp.int32)

            def sg_body(sg, _):
                col0 = sg * (4 * L)
                word = jnp.zeros((L,), jnp.int32)
                for i in range(4):
                    idx = col0 + 4 * iota + i
                    xv = plsc.load_gather(xbuf, [rsplat, idx])
                    num = plsc.load_gather(ntab, [xv])
                    dv = plsc.load_gather(den, [idx])
                    q = num.astype(jnp.float32) / dv
                    y = jnp.minimum(q, 255.0).astype(jnp.int32)
                    word = word | (y << (8 * i))
                outbuf[rr, pl.ds(sg * L, L)] = word
                return 0

            lax.fori_loop(0, C // (4 * L), sg_body, 0)
            return 0

        lax.fori_loop(0, RB, row_body, 0)
        pltpu.sync_copy(outbuf, out_hbm.at[pl.ds(row0, RB)])
        return 0

    lax.fori_loop(0, NBATCH, batch_body, 0)


def kernel(x, denominator_element_table, numerator_table):
    den = _den_kernel(x, denominator_element_table)
    words = _main_kernel(x, numerator_table, den)
    return lax.bitcast_convert_type(words, jnp.uint8).reshape(N, C)


# speedup vs baseline: 476.5155x; 476.5155x over previous
import functools

import jax
import jax.numpy as jnp
from jax import lax
from jax.experimental import pallas as pl
from jax.experimental.pallas import tpu as pltpu
from jax.experimental.pallas import tpu_sc as plsc

NC = 2
NS = 16
L = 16
NW = NC * NS

N = 4096
C = 4096
RPW = N // NW
RB = 8

_CP = pltpu.CompilerParams(needs_layout_passes=False)


def _worker_id():
    return lax.axis_index("s") * NC + lax.axis_index("c")


_mesh = plsc.VectorSubcoreMesh(core_axis_name="c", subcore_axis_name="s")


@functools.partial(
    pl.kernel,
    out_type=jax.ShapeDtypeStruct((N,), jnp.float32),
    mesh=_mesh,
    compiler_params=_CP,
    scratch_types=[
        pltpu.VMEM((L, C), jnp.int32),
        pltpu.VMEM((256,), jnp.int32),
        pltpu.VMEM((RPW,), jnp.float32),
    ],
)
def _den_kernel(x_hbm, dtab_hbm, den_hbm, xbuf, dtab, denout):
    wid = _worker_id()
    base_row = wid * RPW
    pltpu.sync_copy(dtab_hbm, dtab)
    lanes = lax.iota(jnp.int32, L)

    def batch_body(b, _):
        pltpu.sync_copy(x_hbm.at[pl.ds(base_row + b * L, L)], xbuf)

        def col_body(c, acc):
            xv = plsc.load_gather(xbuf, [lanes, jnp.full((L,), c, jnp.int32)])
            return acc + plsc.load_gather(dtab, [xv])

        acc = lax.fori_loop(0, C, col_body, jnp.zeros((L,), jnp.int32),
                            unroll=8)
        denout[pl.ds(b * L, L)] = acc.astype(jnp.float32)
        return 0

    lax.fori_loop(0, RPW // L, batch_body, 0)
    pltpu.sync_copy(denout, den_hbm.at[pl.ds(base_row, RPW)])


@functools.partial(
    pl.kernel,
    out_type=jax.ShapeDtypeStruct((N, C // 4), jnp.int32),
    mesh=_mesh,
    compiler_params=_CP,
    scratch_types=[
        pltpu.VMEM((RB, C), jnp.int32),
        pltpu.VMEM((256,), jnp.int32),
        pltpu.VMEM((N,), jnp.float32),
        pltpu.VMEM((RB, C // 4), jnp.int32),
    ],
)
def _main_kernel(x_hbm, ntab_hbm, den_hbm, out_hbm, xbuf, ntab, den, outbuf):
    wid = _worker_id()
    base_row = wid * RPW
    pltpu.sync_copy(ntab_hbm, ntab)
    pltpu.sync_copy(den_hbm, den)
    iota4 = lax.iota(jnp.int32, L) * 4

    def batch_body(b, _):
        row0 = base_row + b * RB
        pltpu.sync_copy(x_hbm.at[pl.ds(row0, RB)], xbuf)

        def row_body(rr, _):
            rsplat = jnp.full((L,), rr, jnp.int32)

            def sg_body(sg, _):
                cbase = iota4 + sg * (4 * L)
                word = jnp.zeros((L,), jnp.int32)
                for i in range(4):
                    cidx = cbase + i
                    xv = plsc.load_gather(xbuf, [rsplat, cidx])
                    num = plsc.load_gather(ntab, [xv])
                    dv = plsc.load_gather(den, [cidx])
                    q = num.astype(jnp.float32) / dv
                    y = jnp.minimum(q, 255.0).astype(jnp.int32)
                    word = word | (y << (8 * i))
                outbuf[rr, pl.ds(sg * L, L)] = word
                return 0

            lax.fori_loop(0, C // (4 * L), sg_body, 0, unroll=2)
            return 0

        lax.fori_loop(0, RB, row_body, 0)
        pltpu.sync_copy(outbuf, out_hbm.at[pl.ds(row0, RB)])
        return 0

    lax.fori_loop(0, RPW // RB, batch_body, 0)


def kernel(x, denominator_element_table, numerator_table):
    den = _den_kernel(x, denominator_element_table)
    words = _main_kernel(x, numerator_table, den)
    return lax.bitcast_convert_type(words, jnp.uint8).reshape(N, C)


# speedup vs baseline: 557.3986x; 1.1697x over previous
import functools

import jax
import jax.numpy as jnp
from jax import lax
from jax.experimental import pallas as pl
from jax.experimental.pallas import tpu as pltpu
from jax.experimental.pallas import tpu_sc as plsc

NC = 2
NS = 16
L = 16
NW = NC * NS

N = 4096
C = 4096
RPW = N // NW
RB = 8

_CP = pltpu.CompilerParams(needs_layout_passes=False)


def _worker_id():
    return lax.axis_index("s") * NC + lax.axis_index("c")


_mesh = plsc.VectorSubcoreMesh(core_axis_name="c", subcore_axis_name="s")


@functools.partial(
    pl.kernel,
    out_type=jax.ShapeDtypeStruct((N,), jnp.float32),
    mesh=_mesh,
    compiler_params=_CP,
    scratch_types=[
        pltpu.VMEM((L, C), jnp.int32),
        pltpu.VMEM((256,), jnp.int32),
        pltpu.VMEM((RPW,), jnp.float32),
    ],
)
def _den_kernel(x_hbm, dtab_hbm, den_hbm, xbuf, dtab, denout):
    wid = _worker_id()
    base_row = wid * RPW
    pltpu.sync_copy(dtab_hbm, dtab)
    lanes = lax.iota(jnp.int32, L)

    def batch_body(b, _):
        pltpu.sync_copy(x_hbm.at[pl.ds(base_row + b * L, L)], xbuf)

        @plsc.parallel_loop(0, C, unroll=8, carry=jnp.zeros((L,), jnp.int32))
        def acc(c, acc_in):
            xv = plsc.load_gather(xbuf, [lanes, jnp.full((L,), c, jnp.int32)])
            return acc_in + plsc.load_gather(dtab, [xv])
        denout[pl.ds(b * L, L)] = acc.astype(jnp.float32)
        return 0

    lax.fori_loop(0, RPW // L, batch_body, 0)
    pltpu.sync_copy(denout, den_hbm.at[pl.ds(base_row, RPW)])


@functools.partial(
    pl.kernel,
    out_type=jax.ShapeDtypeStruct((N, C // 4), jnp.int32),
    mesh=_mesh,
    compiler_params=_CP,
    scratch_types=[
        pltpu.VMEM((RB, C), jnp.int32),
        pltpu.VMEM((256,), jnp.int32),
        pltpu.VMEM((N,), jnp.float32),
        pltpu.VMEM((RB, C // 4), jnp.int32),
    ],
)
def _main_kernel(x_hbm, ntab_hbm, den_hbm, out_hbm, xbuf, ntab, den, outbuf):
    wid = _worker_id()
    base_row = wid * RPW
    pltpu.sync_copy(ntab_hbm, ntab)
    pltpu.sync_copy(den_hbm, den)
    iota4 = lax.iota(jnp.int32, L) * 4

    def batch_body(b, _):
        row0 = base_row + b * RB
        pltpu.sync_copy(x_hbm.at[pl.ds(row0, RB)], xbuf)

        def row_body(rr, _):
            rsplat = jnp.full((L,), rr, jnp.int32)

            @plsc.parallel_loop(0, C // (4 * L), unroll=2)
            def _(sg):
                cbase = iota4 + sg * (4 * L)
                word = jnp.zeros((L,), jnp.int32)
                for i in range(4):
                    cidx = cbase + i
                    xv = plsc.load_gather(xbuf, [rsplat, cidx])
                    num = plsc.load_gather(ntab, [xv])
                    dv = plsc.load_gather(den, [cidx])
                    q = num.astype(jnp.float32) / dv
                    y = jnp.minimum(q, 255.0).astype(jnp.int32)
                    word = word | (y << (8 * i))
                outbuf[rr, pl.ds(sg * L, L)] = word

            return 0

        lax.fori_loop(0, RB, row_body, 0)
        pltpu.sync_copy(outbuf, out_hbm.at[pl.ds(row0, RB)])
        return 0

    lax.fori_loop(0, RPW // RB, batch_body, 0)


def kernel(x, denominator_element_table, numerator_table):
    den = _den_kernel(x, denominator_element_table)
    words = _main_kernel(x, numerator_table, den)
    return lax.bitcast_convert_type(words, jnp.uint8).reshape(N, C)


# speedup vs baseline: 629.2442x; 1.1289x over previous
import functools

import jax
import jax.numpy as jnp
from jax import lax
from jax.experimental import pallas as pl
from jax.experimental.pallas import tpu as pltpu
from jax.experimental.pallas import tpu_sc as plsc

NC = 2
NS = 16
L = 16
NW = NC * NS

N = 4096
C = 4096
RPW = N // NW
HC = C // 2
RB = 8
NB = RPW // RB
SG = C // (4 * L)

_CP = pltpu.CompilerParams(needs_layout_passes=False)


def _worker_id():
    return lax.axis_index("s") * NC + lax.axis_index("c")


_mesh = plsc.VectorSubcoreMesh(core_axis_name="c", subcore_axis_name="s")


@functools.partial(
    pl.kernel,
    out_type=jax.ShapeDtypeStruct((N,), jnp.float32),
    mesh=_mesh,
    compiler_params=_CP,
    scratch_types=[
        pltpu.VMEM((2, L, HC), jnp.int32),
        pltpu.VMEM((256,), jnp.int32),
        pltpu.VMEM((RPW,), jnp.float32),
        pltpu.SemaphoreType.DMA,
        pltpu.SemaphoreType.DMA,
    ],
)
def _den_kernel(x_hbm, dtab_hbm, den_hbm, xbuf, dtab, denout, sem0, sem1):
    wid = _worker_id()
    base_row = wid * RPW
    pltpu.sync_copy(dtab_hbm, dtab)
    lanes = lax.iota(jnp.int32, L)
    sems = (sem0, sem1)

    def chunk_src(g, ch):
        rows = base_row + g * L
        return x_hbm.at[pl.ds(rows, L), pl.ds(ch * HC, HC)]

    pltpu.async_copy(chunk_src(0, 0), xbuf.at[0], sem0)

    def grp_body(g, _):
        acc_g = jnp.zeros((L,), jnp.int32)
        for ch in (0, 1):
            sem = sems[ch]
            pltpu.make_async_copy(chunk_src(g, ch), xbuf.at[ch], sem).wait()
            if ch == 0:
                pltpu.async_copy(chunk_src(g, 1), xbuf.at[1], sem1)
            else:
                @pl.when(g + 1 < RPW // L)
                def _():
                    pltpu.async_copy(chunk_src(g + 1, 0), xbuf.at[0], sem0)

            @plsc.parallel_loop(0, HC, unroll=8, carry=acc_g)
            def acc_g(c, acc_in):
                xv = plsc.load_gather(
                    xbuf.at[ch], [lanes, jnp.full((L,), c, jnp.int32)])
                return acc_in + plsc.load_gather(dtab, [xv])

        denout[pl.ds(g * L, L)] = acc_g.astype(jnp.float32)
        return 0

    lax.fori_loop(0, RPW // L, grp_body, 0)
    pltpu.sync_copy(denout, den_hbm.at[pl.ds(base_row, RPW)])


@functools.partial(
    pl.kernel,
    out_type=jax.ShapeDtypeStruct((N, C // 4), jnp.int32),
    mesh=_mesh,
    compiler_params=_CP,
    scratch_types=[
        pltpu.VMEM((2, RB, C), jnp.int32),
        pltpu.VMEM((256,), jnp.int32),
        pltpu.VMEM((256,), jnp.float32),
        pltpu.VMEM((N,), jnp.float32),
        pltpu.VMEM((2, RB, C // 4), jnp.int32),
        pltpu.SemaphoreType.DMA,
        pltpu.SemaphoreType.DMA,
        pltpu.SemaphoreType.DMA,
        pltpu.SemaphoreType.DMA,
    ],
)
def _main_kernel(x_hbm, ntab_hbm, den_hbm, out_hbm,
                 xbuf, ntab, ntab_f, den, outbuf,
                 sin0, sin1, sout0, sout1):
    wid = _worker_id()
    base_row = wid * RPW
    pltpu.sync_copy(ntab_hbm, ntab)
    pltpu.sync_copy(den_hbm, den)
    sin = (sin0, sin1)
    sout = (sout0, sout1)
    iota4 = lax.iota(jnp.int32, L) * 4

    @plsc.parallel_loop(0, 256 // L)
    def _(k):
        ntab_f[pl.ds(k * L, L)] = ntab[pl.ds(k * L, L)].astype(jnp.float32)

    def in_src(b):
        return x_hbm.at[pl.ds(base_row + b * RB, RB)]

    def out_dst(b):
        return out_hbm.at[pl.ds(base_row + b * RB, RB)]

    pltpu.async_copy(in_src(0), xbuf.at[0], sin0)

    def half_body(h, _):
        for s in (0, 1):
            b = h * 2 + s
            pltpu.make_async_copy(in_src(b), xbuf.at[s], sin[s]).wait()

            @pl.when(b + 1 < NB)
            def _():
                pltpu.async_copy(in_src(b + 1), xbuf.at[1 - s], sin[1 - s])

            @pl.when(b >= 2)
            def _():
                pltpu.make_async_copy(outbuf.at[s], out_dst(b - 2),
                                      sout[s]).wait()

            xb = xbuf.at[s]
            ob = outbuf.at[s]

            @plsc.parallel_loop(0, RB * SG, unroll=2)
            def _(t):
                rr = t >> 6
                cbase = iota4 + ((t & (SG - 1)) << 6)
                rsplat = jnp.full((L,), rr, jnp.int32)
                word = jnp.zeros((L,), jnp.int32)
                for i in range(4):
                    cidx = cbase + i
                    xv = plsc.load_gather(xb, [rsplat, cidx])
                    num = plsc.load_gather(ntab_f, [xv])
                    dv = plsc.load_gather(den, [cidx])
                    y = jnp.minimum(num / dv, 255.0).astype(jnp.int32)
                    word = word | (y << (8 * i))
                ob[rr, pl.ds((t & (SG - 1)) * L, L)] = word

            pltpu.async_copy(outbuf.at[s], out_dst(b), sout[s])
        return 0

    lax.fori_loop(0, NB // 2, half_body, 0)
    pltpu.make_async_copy(outbuf.at[0], out_dst(NB - 2), sout0).wait()
    pltpu.make_async_copy(outbuf.at[1], out_dst(NB - 1), sout1).wait()


def kernel(x, denominator_element_table, numerator_table):
    den = _den_kernel(x, denominator_element_table)
    words = _main_kernel(x, numerator_table, den)
    return lax.bitcast_convert_type(words, jnp.uint8).reshape(N, C)


# speedup vs baseline: 960.9163x; 1.5271x over previous
import functools

import jax
import jax.numpy as jnp
from jax import lax
from jax.experimental import pallas as pl
from jax.experimental.pallas import tpu as pltpu
from jax.experimental.pallas import tpu_sc as plsc

NC = 2
NS = 16
L = 16
NW = NC * NS

N = 4096
C = 4096
RPW = N // NW
HC = C // 2
RB = 8
NB = RPW // RB
SG = C // (4 * L)

_CP = pltpu.CompilerParams(needs_layout_passes=False)


def _worker_id():
    return lax.axis_index("s") * NC + lax.axis_index("c")


_mesh = plsc.VectorSubcoreMesh(core_axis_name="c", subcore_axis_name="s")


@functools.partial(
    pl.kernel,
    out_type=jax.ShapeDtypeStruct((N,), jnp.float32),
    mesh=_mesh,
    compiler_params=_CP,
    scratch_types=[
        pltpu.VMEM((2, L, HC), jnp.int32),
        pltpu.VMEM((256,), jnp.int32),
        pltpu.VMEM((256 * L,), jnp.int32),
        pltpu.VMEM((RPW,), jnp.float32),
        pltpu.SemaphoreType.DMA,
        pltpu.SemaphoreType.DMA,
    ],
)
def _den_kernel(x_hbm, dtab_hbm, den_hbm, xbuf, dtab, dtab_rep, denout,
                sem0, sem1):
    wid = _worker_id()
    base_row = wid * RPW
    pltpu.sync_copy(dtab_hbm, dtab)
    lanes = lax.iota(jnp.int32, L)
    sems = (sem0, sem1)

    @plsc.parallel_loop(0, 256, unroll=4)
    def _(v):
        dtab_rep[pl.ds(v * L, L)] = plsc.load_gather(
            dtab, [jnp.full((L,), v, jnp.int32)])

    def chunk_src(g, ch):
        rows = base_row + g * L
        return x_hbm.at[pl.ds(rows, L), pl.ds(ch * HC, HC)]

    pltpu.async_copy(chunk_src(0, 0), xbuf.at[0], sem0)

    def grp_body(g, _):
        acc_g = jnp.zeros((L,), jnp.int32)
        for ch in (0, 1):
            sem = sems[ch]
            pltpu.make_async_copy(chunk_src(g, ch), xbuf.at[ch], sem).wait()
            if ch == 0:
                pltpu.async_copy(chunk_src(g, 1), xbuf.at[1], sem1)
            else:
                @pl.when(g + 1 < RPW // L)
                def _():
                    pltpu.async_copy(chunk_src(g + 1, 0), xbuf.at[0], sem0)

            @plsc.parallel_loop(0, HC, unroll=8, carry=acc_g)
            def acc_g(c, acc_in):
                cvec = (lanes + c) & (HC - 1)
                xv = plsc.load_gather(xbuf.at[ch], [lanes, cvec])
                dt = plsc.load_gather(dtab_rep, [(xv << 4) | lanes])
                return acc_in + dt

        denout[pl.ds(g * L, L)] = acc_g.astype(jnp.float32)
        return 0

    lax.fori_loop(0, RPW // L, grp_body, 0)
    pltpu.sync_copy(denout, den_hbm.at[pl.ds(base_row, RPW)])


@functools.partial(
    pl.kernel,
    out_type=jax.ShapeDtypeStruct((N, C // 4), jnp.int32),
    mesh=_mesh,
    compiler_params=_CP,
    scratch_types=[
        pltpu.VMEM((2, RB, C), jnp.int32),
        pltpu.VMEM((256,), jnp.int32),
        pltpu.VMEM((256 * L,), jnp.float32),
        pltpu.VMEM((N,), jnp.float32),
        pltpu.VMEM((N,), jnp.float32),
        pltpu.VMEM((2, RB, C // 4), jnp.int32),
        pltpu.SemaphoreType.DMA,
        pltpu.SemaphoreType.DMA,
        pltpu.SemaphoreType.DMA,
        pltpu.SemaphoreType.DMA,
    ],
)
def _main_kernel(x_hbm, ntab_hbm, den_hbm, out_hbm,
                 xbuf, ntab, ntab_f, den, den_p, outbuf,
                 sin0, sin1, sout0, sout1):
    wid = _worker_id()
    base_row = wid * RPW
    pltpu.sync_copy(ntab_hbm, ntab)
    pltpu.sync_copy(den_hbm, den)
    sin = (sin0, sin1)
    sout = (sout0, sout1)
    lanes = lax.iota(jnp.int32, L)
    iota4 = lanes * 4

    @plsc.parallel_loop(0, 256, unroll=4)
    def _(v):
        nv = plsc.load_gather(ntab, [jnp.full((L,), v, jnp.int32)])
        ntab_f[pl.ds(v * L, L)] = nv.astype(jnp.float32)

    @plsc.parallel_loop(0, N // L, unroll=4)
    def _(h):
        src = ((h >> 2) << 6) + iota4 + (h & 3)
        den_p[pl.ds(h * L, L)] = plsc.load_gather(den, [src])

    def in_src(b):
        return x_hbm.at[pl.ds(base_row + b * RB, RB)]

    def out_dst(b):
        return out_hbm.at[pl.ds(base_row + b * RB, RB)]

    pltpu.async_copy(in_src(0), xbuf.at[0], sin0)

    def half_body(h, _):
        for s in (0, 1):
            b = h * 2 + s
            pltpu.make_async_copy(in_src(b), xbuf.at[s], sin[s]).wait()

            @pl.when(b + 1 < NB)
            def _():
                pltpu.async_copy(in_src(b + 1), xbuf.at[1 - s], sin[1 - s])

            @pl.when(b >= 2)
            def _():
                pltpu.make_async_copy(outbuf.at[s], out_dst(b - 2),
                                      sout[s]).wait()

            xb = xbuf.at[s]
            ob = outbuf.at[s]

            @plsc.parallel_loop(0, RB * SG, unroll=2)
            def _(t):
                rr = t >> 6
                sg = t & (SG - 1)
                cbase = iota4 + (sg << 6)
                rsplat = jnp.full((L,), rr, jnp.int32)
                word = jnp.zeros((L,), jnp.int32)
                for i in range(4):
                    xv = plsc.load_gather(xb, [rsplat, cbase + i])
                    num = plsc.load_gather(ntab_f, [(xv << 4) | lanes])
                    dv = den_p[pl.ds((sg * 4 + i) * L, L)]
                    y = jnp.minimum(num / dv, 255.0).astype(jnp.int32)
                    word = word | (y << (8 * i))
                ob[rr, pl.ds(sg * L, L)] = word

            pltpu.async_copy(outbuf.at[s], out_dst(b), sout[s])
        return 0

    lax.fori_loop(0, NB // 2, half_body, 0)
    pltpu.make_async_copy(outbuf.at[0], out_dst(NB - 2), sout0).wait()
    pltpu.make_async_copy(outbuf.at[1], out_dst(NB - 1), sout1).wait()


def kernel(x, denominator_element_table, numerator_table):
    den = _den_kernel(x, denominator_element_table)
    words = _main_kernel(x, numerator_table, den)
    return lax.bitcast_convert_type(words, jnp.uint8).reshape(N, C)


# speedup vs baseline: 1944.8496x; 2.0240x over previous
import functools

import jax
import jax.numpy as jnp
from jax import lax
from jax.experimental import pallas as pl
from jax.experimental.pallas import tpu as pltpu
from jax.experimental.pallas import tpu_sc as plsc

NC = 2
NS = 16
L = 16
NW = NC * NS

N = 4096
C = 4096
RPW = N // NW
HC = C // 2
RB = 8
NB = RPW // RB
SG = C // (4 * L)

_CP = pltpu.CompilerParams(needs_layout_passes=False)


def _worker_id():
    return lax.axis_index("s") * NC + lax.axis_index("c")


_mesh = plsc.VectorSubcoreMesh(core_axis_name="c", subcore_axis_name="s")


@functools.partial(
    pl.kernel,
    out_type=jax.ShapeDtypeStruct((N,), jnp.float32),
    mesh=_mesh,
    compiler_params=_CP,
    scratch_types=[
        pltpu.VMEM((2, L, HC), jnp.int32),
        pltpu.VMEM((256,), jnp.int32),
        pltpu.VMEM((256 * L,), jnp.int32),
        pltpu.VMEM((RPW,), jnp.float32),
        pltpu.SemaphoreType.DMA,
        pltpu.SemaphoreType.DMA,
    ],
)
def _den_kernel(x_hbm, dtab_hbm, den_hbm, xbuf, dtab, dtab_rep, denout,
                sem0, sem1):
    wid = _worker_id()
    base_row = wid * RPW
    pltpu.sync_copy(dtab_hbm, dtab)
    lanes = lax.iota(jnp.int32, L)
    sems = (sem0, sem1)

    @plsc.parallel_loop(0, 256, unroll=4)
    def _(v):
        dtab_rep[pl.ds(v * L, L)] = plsc.load_gather(
            dtab, [jnp.full((L,), v, jnp.int32)])

    def chunk_src(g, ch):
        rows = base_row + g * L
        return x_hbm.at[pl.ds(rows, L), pl.ds(ch * HC, HC)]

    pltpu.async_copy(chunk_src(0, 0), xbuf.at[0], sem0)

    def grp_body(g, _):
        acc_g = jnp.zeros((L,), jnp.int32)
        for ch in (0, 1):
            sem = sems[ch]
            pltpu.make_async_copy(chunk_src(g, ch), xbuf.at[ch], sem).wait()
            if ch == 0:
                pltpu.async_copy(chunk_src(g, 1), xbuf.at[1], sem1)
            else:
                @pl.when(g + 1 < RPW // L)
                def _():
                    pltpu.async_copy(chunk_src(g + 1, 0), xbuf.at[0], sem0)

            @plsc.parallel_loop(0, HC, unroll=8, carry=acc_g)
            def acc_g(c, acc_in):
                cvec = (lanes + c) & (HC - 1)
                xv = plsc.load_gather(xbuf.at[ch], [lanes, cvec])
                dt = plsc.load_gather(dtab_rep, [(xv << 4) | lanes])
                return acc_in + dt

        denout[pl.ds(g * L, L)] = acc_g.astype(jnp.float32)
        return 0

    lax.fori_loop(0, RPW // L, grp_body, 0)
    pltpu.sync_copy(denout, den_hbm.at[pl.ds(base_row, RPW)])


WPB = RB // 4
BLK = 8
BPB = BLK // WPB


@functools.partial(
    pl.kernel,
    out_type=jax.ShapeDtypeStruct((N, C), jnp.uint8),
    mesh=_mesh,
    compiler_params=_CP,
    scratch_types=[
        pltpu.VMEM((2, RB, C), jnp.int32),
        pltpu.VMEM((256,), jnp.int32),
        pltpu.VMEM((256 * L,), jnp.float32),
        pltpu.VMEM((N,), jnp.float32),
        pltpu.VMEM((BLK, C), jnp.int32),
        pltpu.SemaphoreType.DMA,
        pltpu.SemaphoreType.DMA,
        pltpu.SemaphoreType.DMA,
    ],
)
def _main_kernel(x_hbm, ntab_hbm, den_hbm, out_hbm,
                 xbuf, ntab, ntab_f, den, outbuf,
                 sin0, sin1, sout):
    wid = _worker_id()
    base_row = wid * RPW
    wout = out_hbm.bitcast(jnp.int32)
    base_wr = wid * (RPW // 4)
    pltpu.sync_copy(ntab_hbm, ntab)
    pltpu.sync_copy(den_hbm, den)
    sin = (sin0, sin1)
    lanes = lax.iota(jnp.int32, L)

    @plsc.parallel_loop(0, 256, unroll=4)
    def _(v):
        nv = plsc.load_gather(ntab, [jnp.full((L,), v, jnp.int32)])
        ntab_f[pl.ds(v * L, L)] = nv.astype(jnp.float32)

    def in_src(b):
        return x_hbm.at[pl.ds(base_row + b * RB, RB)]

    def out_dst(blk):
        return wout.at[pl.ds(base_wr + blk * BLK, BLK)]

    pltpu.async_copy(in_src(0), xbuf.at[0], sin0)

    for b in range(NB):
        s = b & 1
        blk, bi = divmod(b, BPB)
        pltpu.make_async_copy(in_src(b), xbuf.at[s], sin[s]).wait()
        if b + 1 < NB:
            pltpu.async_copy(in_src(b + 1), xbuf.at[1 - s], sin[1 - s])
        if bi == 0 and blk > 0:
            pltpu.make_async_copy(outbuf, out_dst(blk - 1), sout).wait()
        xb = xbuf.at[s]

        @plsc.parallel_loop(0, WPB * (C // L), unroll=2)
        def _(t):
            wr = t >> 8
            cg = t & (C // L - 1)
            dv = den[pl.ds(cg * L, L)]
            word = jnp.zeros((L,), jnp.int32)
            for r in range(4):
                xv = xb[wr * 4 + r, pl.ds(cg * L, L)]
                num = plsc.load_gather(ntab_f, [(xv << 4) | lanes])
                y = jnp.minimum(num / dv, 255.0).astype(jnp.int32)
                word = word | (y << (8 * r))
            outbuf[bi * WPB + wr, pl.ds(cg * L, L)] = word

        if bi == BPB - 1:
            pltpu.async_copy(outbuf, out_dst(blk), sout)

    pltpu.make_async_copy(outbuf, out_dst(NB // BPB - 1), sout).wait()


def kernel(x, denominator_element_table, numerator_table):
    den = _den_kernel(x, denominator_element_table)
    return _main_kernel(x, numerator_table, den)
